# baseline (speedup 1.0000x reference)
import functools

import jax
import jax.numpy as jnp
from jax import lax
from jax.experimental import pallas as pl
from jax.experimental.pallas import tpu as pltpu
from jax.experimental.pallas import tpu_sc as plsc

EMB = 10
LANES = 16


@functools.lru_cache(maxsize=None)
def _build(B, C):
    info = plsc.get_sparse_core_info()
    nw = info.num_cores * info.num_subcores
    assert B % (8 * nw) == 0
    rpw = B // nw
    in_w = rpw * C
    out_w = rpw * EMB
    assert out_w % 80 == 0
    n_outer = out_w // 80

    mesh = plsc.VectorSubcoreMesh(core_axis_name="c", subcore_axis_name="s")

    @functools.partial(
        pl.kernel,
        mesh=mesh,
        out_type=jax.ShapeDtypeStruct((B * EMB,), jnp.int32),
        scratch_types=[
            pltpu.VMEM((in_w,), jnp.int32),
            pltpu.VMEM((out_w,), jnp.int32),
        ],
        compiler_params=pltpu.CompilerParams(needs_layout_passes=False),
    )
    def run(in_hbm, out_hbm, blk_v, out_v):
        wid = lax.axis_index("s") * info.num_cores + lax.axis_index("c")
        pltpu.sync_copy(in_hbm.at[pl.ds(wid * in_w, in_w)], blk_v)

        iota = lax.iota(jnp.int32, LANES)
        offs = [((LANES * p + iota) // EMB) * C for p in range(5)]

        def body(q, carry):
            row_off = q * 8 * C
            for p in range(5):
                val = plsc.load_gather(blk_v, [offs[p] + row_off])
                out_v[pl.ds(q * 80 + p * LANES, LANES)] = val
            return carry

        lax.fori_loop(0, n_outer, body, 0, unroll=4)

        pltpu.sync_copy(out_v, out_hbm.at[pl.ds(wid * out_w, out_w)])

    return run


def kernel(inputs):
    B, C = inputs.shape
    flat = inputs.astype(jnp.int32).reshape(B * C)
    out = _build(B, C)(flat)
    return out.reshape(B, EMB)
